# baseline (speedup 1.0000x reference)
import functools

import jax
import jax.numpy as jnp
from jax import lax
from jax.experimental import pallas as pl
from jax.experimental.pallas import tpu as pltpu, tpu_sc as plsc


def kernel(features, labels):
    del labels
    n, h, w = features.shape
    rows, cols = n * h, w
    flat = features.reshape(rows, cols)
    info = plsc.get_sparse_core_info()
    nc, ns = info.num_cores, info.num_subcores
    nw = nc * ns
    rows_pw = rows // nw
    mesh = plsc.VectorSubcoreMesh(core_axis_name="c", subcore_axis_name="s")

    @functools.partial(
        pl.kernel,
        mesh=mesh,
        out_type=jax.ShapeDtypeStruct((rows, cols), flat.dtype),
    )
    def k(x_hbm, o_hbm):
        wid = lax.axis_index("s") * nc + lax.axis_index("c")
        base = wid * rows_pw
        pltpu.sync_copy(
            x_hbm.at[pl.ds(base, rows_pw)], o_hbm.at[pl.ds(base, rows_pw)]
        )

    return k(flat).reshape(n, h, w)


# speedup vs baseline: 49.5628x; 49.5628x over previous
import jax
import jax.numpy as jnp
from jax.experimental import pallas as pl


def _copy_block(x_ref, o_ref):
    o_ref[...] = x_ref[...]


def kernel(features, labels):
    del labels
    n, h, w = features.shape
    rows, cols = n * h, w
    flat = features.reshape(rows, cols)
    block_rows = 16384
    out = pl.pallas_call(
        _copy_block,
        grid=(rows // block_rows,),
        in_specs=[pl.BlockSpec((block_rows, cols), lambda i: (i, 0))],
        out_specs=pl.BlockSpec((block_rows, cols), lambda i: (i, 0)),
        out_shape=jax.ShapeDtypeStruct((rows, cols), features.dtype),
    )(flat)
    return out.reshape(n, h, w)
